# Initial kernel scaffold; baseline (speedup 1.0000x reference)
import jax
import jax.numpy as jnp
from jax.experimental import pallas as pl


def kernel(x, h, t, num_atoms_per_ligand, batch_ligand, params, time_table):
    raise NotImplementedError("write your pallas kernel here")


# speedup vs baseline: 6.4172x; 6.4172x over previous
import math

import jax
import jax.numpy as jnp
from jax import lax
from jax.experimental import pallas as pl

N = 50000
AT = 50
B = 1000
K = 8
IN_F = 16
TEMB = 16
HID = 128
OUT_F = 64
NG = 20
DEPTH = 2
NT = 1000

G = 8
V = G * AT
E = V * K
NBLK = B // G

_f32 = jnp.float32


def _mm(a, b):
    return lax.dot_general(a, b, (((1,), (0,)), ((), ())),
                           preferred_element_type=_f32)


def _mmT(a, b):
    return lax.dot_general(a, b, (((1,), (1,)), ((), ())),
                           preferred_element_type=_f32)


_bf16 = jnp.bfloat16


def _mm_sel(oh, vals):
    vh = vals.astype(_bf16)
    vl = (vals - vh.astype(_f32)).astype(_bf16)
    ohb = oh.astype(_bf16)
    dn = (((1,), (0,)), ((), ()))
    return (lax.dot_general(ohb, vh, dn, preferred_element_type=_f32)
            + lax.dot_general(ohb, vl, dn, preferred_element_type=_f32))


def _mmT3(a, b):
    ah = a.astype(_bf16)
    al = (a - ah.astype(_f32)).astype(_bf16)
    bh = b.astype(_bf16)
    bl = (b - bh.astype(_f32)).astype(_bf16)
    dn = (((1,), (1,)), ((), ()))
    return (lax.dot_general(ah, bh, dn, preferred_element_type=_f32)
            + lax.dot_general(ah, bl, dn, preferred_element_type=_f32)
            + lax.dot_general(al, bh, dn, preferred_element_type=_f32))


def _silu(v):
    return v * jax.nn.sigmoid(v)


def _ln(v, g, b):
    m = jnp.mean(v, axis=-1, keepdims=True)
    d = v - m
    var = jnp.mean(d * d, axis=-1, keepdims=True)
    return d * jax.lax.rsqrt(var + 1e-5) * g + b


def _body(*refs):
    xb = refs[0][...]
    hb = refs[1][...]
    tfb = refs[2][...]
    tt = refs[3][...]
    na = refs[4][0, 0]
    w = [r[...] for r in refs[5:-1]]
    out_ref = refs[-1]

    iota_r = lax.broadcasted_iota(jnp.int32, (V, V), 0).astype(_f32)
    iota_c = lax.broadcasted_iota(jnp.int32, (V, V), 1).astype(_f32)
    lig_r = jnp.floor((iota_r + 0.5) * (1.0 / AT))
    lig_c = jnp.floor((iota_c + 0.5) * (1.0 / AT))

    iott = lax.broadcasted_iota(jnp.int32, (V, NT), 1).astype(_f32)
    temb = _mm_sel((iott == tfb).astype(_f32), tt)

    rcol = jnp.sum(xb * xb, axis=1, keepdims=True)
    am = jnp.concatenate([-2.0 * xb, jnp.ones((V, 1), _f32)], axis=1)
    bm = jnp.concatenate([xb, rcol], axis=1)
    d2 = _mmT3(am, bm) + rcol
    d2m = (d2 + jnp.where(iota_r == iota_c, 1e9, 0.0)
           + jnp.where(lig_r == lig_c, 0.0, 1e30))

    ohs_list, rad_list = [], []
    for _ in range(K):
        minv = jnp.min(d2m, axis=1, keepdims=True)
        idx = jnp.min(jnp.where(d2m == minv, iota_c, 1e9),
                      axis=1, keepdims=True)
        sel = iota_c == idx
        ohs_list.append(sel.astype(_f32))
        rad_list.append(minv)
        d2m = jnp.where(sel, 1e30, d2m)
    ohs = jnp.concatenate(ohs_list, axis=0)
    radial = jnp.concatenate(rad_list, axis=0)
    keepf = (radial < 7.0).astype(_f32)

    c = math.log(5.0) / (NG - 1)
    i20 = lax.broadcasted_iota(jnp.int32, (1, NG), 1).astype(_f32)
    off = jnp.exp(c * i20) - 1.0
    jj = jnp.maximum(i20, 1.0)
    df = jnp.exp(c * jj) - jnp.exp(c * (jj - 1.0))
    coeff = -0.5 / (df * df)
    dc = jnp.clip(radial, 0.0, 4.0) - off
    smear = jnp.exp(coeff * dc * dc)

    it = iter(w)
    emb_h_w, emb_t_w, emb_b = next(it), next(it), next(it)
    z = _mm(hb, emb_h_w) + _mm(temb, emb_t_w) + emb_b

    for _ in range(DEPTH):
        (w_row, w_col, w_sm, w_tb, e1_b, ln1_g, ln1_b, e2_w, e2_b,
         att_w, att_b, n1a, n1b, n1_b, ln2_g, ln2_b, n2_w, n2_b) = (
            next(it) for _ in range(18))
        common = _mm(z, w_row) + _mm(temb, w_tb) + e1_b
        gath = _mm_sel(ohs, _mm(z, w_col))
        m = (jnp.concatenate([common] * K, axis=0) + gath
             + _mm(smear, w_sm))
        m = _silu(_ln(m, ln1_g, ln1_b))
        mij = _silu(_mm(m, e2_w) + e2_b)
        att = jax.nn.sigmoid(_mm(mij, att_w) + att_b)
        ef = mij * att * keepf
        agg = ef[0:V]
        for k in range(1, K):
            agg = agg + ef[k * V:(k + 1) * V]
        agg = agg * 0.2
        o = _silu(_ln(_mm(z, n1a) + _mm(agg, n1b) + n1_b, ln2_g, ln2_b))
        z = z + _mm(o, n2_w) + n2_b

    emb_out_w, emb_out_b, out_w, out_b = (next(it) for _ in range(4))
    zo = _mm(z, emb_out_w) + emb_out_b
    pl_c = jnp.floor((lax.broadcasted_iota(jnp.int32, (G, V), 1)
                      .astype(_f32) + 0.5) * (1.0 / AT))
    pmat = (lax.broadcasted_iota(jnp.int32, (G, V), 0).astype(_f32)
            == pl_c).astype(_f32)
    pooled = _mm_sel(pmat, zo)
    out_ref[...] = _mm(pooled, out_w) / na + out_b


def kernel(x, h, t, num_atoms_per_ligand, batch_ligand, params, time_table):
    xr = x.reshape(NBLK, G, AT, 3)
    eye_g = jnp.eye(G, dtype=_f32)
    xblk = jnp.einsum('ngac,gh->ngahc', xr, eye_g).reshape(N, 3 * G)
    tf = t.astype(_f32).reshape(N, 1)
    na = jnp.asarray(num_atoms_per_ligand, _f32).reshape(1, 1)

    p = params
    weights = [p['emb_in_W'][:IN_F], p['emb_in_W'][IN_F:],
               p['emb_in_b'].reshape(1, HID)]
    for lp in p['layers']:
        weights += [
            lp['e1_W'][0:HID], lp['e1_W'][HID:2 * HID],
            lp['e1_W'][2 * HID:2 * HID + NG], lp['e1_W'][2 * HID + NG:],
            lp['e1_b'].reshape(1, HID),
            lp['ln1_g'].reshape(1, HID), lp['ln1_b'].reshape(1, HID),
            lp['e2_W'], lp['e2_b'].reshape(1, HID),
            lp['att_W'], lp['att_b'].reshape(1, 1),
            lp['n1_W'][:HID], lp['n1_W'][HID:],
            lp['n1_b'].reshape(1, HID),
            lp['ln2_g'].reshape(1, HID), lp['ln2_b'].reshape(1, HID),
            lp['n2_W'], lp['n2_b'].reshape(1, HID),
        ]
    weights += [p['emb_out_W'], p['emb_out_b'].reshape(1, OUT_F),
                p['out_W'], p['out_b'].reshape(1, 1)]

    data_specs = [
        pl.BlockSpec((V, 3 * G), lambda i: (i, 0)),
        pl.BlockSpec((V, IN_F), lambda i: (i, 0)),
        pl.BlockSpec((V, 1), lambda i: (i, 0)),
        pl.BlockSpec((NT, TEMB), lambda i: (0, 0)),
        pl.BlockSpec((1, 1), lambda i: (0, 0)),
    ]
    w_specs = [pl.BlockSpec(w.shape, lambda i: (0, 0)) for w in weights]

    out = pl.pallas_call(
        _body,
        grid=(NBLK,),
        in_specs=data_specs + w_specs,
        out_specs=pl.BlockSpec((G, 1), lambda i: (i, 0)),
        out_shape=jax.ShapeDtypeStruct((B, 1), _f32),
    )(xblk, h.astype(_f32), tf, time_table, na, *weights)
    return out
